# baseline (speedup 1.0000x reference)
import functools
import math

import jax
import jax.numpy as jnp
from jax import lax
from jax.experimental import pallas as pl
from jax.experimental.pallas import tpu as pltpu
from jax.experimental.pallas import tpu_sc as plsc

D = 64
SEQ = 200
HALF = SEQ // 2
MAX_WAVELENGTH = 10000.0
L = 16
NC, NS = 2, 16
NW = NC * NS
DP = 128
NBUF = 2
JU = 4


def _pe_tc_kernel(o_ref):
    pos = lax.broadcasted_iota(jnp.int32, (SEQ, D), 0).astype(jnp.float32)
    d_idx = lax.broadcasted_iota(jnp.int32, (SEQ, D), 1)
    half = ((d_idx // 2) * 2).astype(jnp.float32)
    ts = jnp.exp(half * jnp.float32(-math.log(MAX_WAVELENGTH) / D))
    angles = pos * ts
    cos_mask = (d_idx % 2).astype(jnp.float32)
    o_ref[...] = jnp.sin(angles) * (1.0 - cos_mask) + jnp.cos(angles) * cos_mask


def _emb_sc(x, pe, table, batch):
    spw = batch // NW
    nround = spw // NBUF
    mesh = plsc.VectorSubcoreMesh(core_axis_name="c", subcore_axis_name="s")

    @functools.partial(
        pl.kernel,
        out_type=jax.ShapeDtypeStruct((batch, SEQ, DP), jnp.float32),
        mesh=mesh,
        compiler_params=pltpu.CompilerParams(use_tc_tiling_on_sc=False),
        scratch_types=[
            pltpu.VMEM((spw, 2, HALF), jnp.int32),
            pltpu.VMEM((NBUF, SEQ, D), jnp.float32),
            pltpu.VMEM((NBUF, SEQ, DP), jnp.float32),
            pltpu.VMEM((SEQ, D), jnp.float32),
        ]
        + [pltpu.SemaphoreType.DMA] * (2 * NBUF),
    )
    def k(x_hbm, pe_hbm, table_hbm, out_hbm, idx_v, rows_v, obuf_v, pe_v, *sems):
        gsem = sems[:NBUF]
        osem = sems[NBUF:]
        wid = lax.axis_index("s") * NC + lax.axis_index("c")
        base = wid * spw
        pltpu.sync_copy(pe_hbm, pe_v)
        pltpu.sync_copy(x_hbm.at[pl.ds(base, spw)], idx_v)

        def issue_gather(i, b):
            pltpu.async_copy(
                table_hbm.at[idx_v.at[i, 0]],
                rows_v.at[b, pl.ds(0, HALF)],
                gsem[b],
            )
            pltpu.async_copy(
                table_hbm.at[idx_v.at[i, 1]],
                rows_v.at[b, pl.ds(HALF, HALF)],
                gsem[b],
            )

        def wait_gather(i, b):
            pltpu.make_async_copy(
                table_hbm.at[idx_v.at[i, 0]],
                rows_v.at[b, pl.ds(0, HALF)],
                gsem[b],
            ).wait()
            pltpu.make_async_copy(
                table_hbm.at[idx_v.at[i, 1]],
                rows_v.at[b, pl.ds(HALF, HALF)],
                gsem[b],
            ).wait()

        def issue_out(i, b):
            pltpu.async_copy(obuf_v.at[b], out_hbm.at[base + i], osem[b])

        def wait_out(i, b):
            pltpu.make_async_copy(obuf_v.at[b], out_hbm.at[base + i], osem[b]).wait()

        issue_gather(0, 0)

        def round_body(r, carry):
            for b in range(NBUF):
                i = r * NBUF + b
                nxt = i + 1
                bn = (b + 1) % NBUF

                @pl.when(jnp.logical_and(nxt >= NBUF, nxt < spw))
                def _():
                    wait_out(nxt - NBUF, bn)

                @pl.when(nxt < spw)
                def _():
                    issue_gather(nxt, bn)

                wait_gather(i, b)

                def jbody(j0, acc):
                    for ju in range(JU):
                        j = j0 * JU + ju
                        for dd in range(D // L):
                            sl = pl.ds(dd * L, L)
                            obuf_v[b, j, sl] = rows_v[b, j, sl] + pe_v[j, sl]
                    return acc

                lax.fori_loop(0, SEQ // JU, jbody, 0)

                issue_out(i, b)
            return carry

        lax.fori_loop(0, nround, round_body, 0)
        for b in range(NBUF):
            wait_out(spw - NBUF + b, b)

    return k(x, pe, table)[:, :, :D]


def kernel(x, table):
    batch, seq = x.shape
    assert seq == SEQ and batch % NW == 0
    xi = x.astype(jnp.int32).reshape(batch, 2, HALF)
    pe = pl.pallas_call(
        _pe_tc_kernel,
        out_shape=jax.ShapeDtypeStruct((SEQ, D), jnp.float32),
    )()
    return _emb_sc(xi, pe, table, batch)


# speedup vs baseline: 1.5139x; 1.5139x over previous
import functools
import math

import jax
import jax.numpy as jnp
from jax import lax
from jax.experimental import pallas as pl
from jax.experimental.pallas import tpu as pltpu
from jax.experimental.pallas import tpu_sc as plsc

D = 64
SEQ = 200
HALF = SEQ // 2
MAX_WAVELENGTH = 10000.0
L = 16
NC, NS = 2, 16
NW = NC * NS
DP = 128
NBUF = 4
JU = 4


def _pe_tc_kernel(o_ref):
    pos = lax.broadcasted_iota(jnp.int32, (SEQ, D), 0).astype(jnp.float32)
    d_idx = lax.broadcasted_iota(jnp.int32, (SEQ, D), 1)
    half = ((d_idx // 2) * 2).astype(jnp.float32)
    ts = jnp.exp(half * jnp.float32(-math.log(MAX_WAVELENGTH) / D))
    angles = pos * ts
    cos_mask = (d_idx % 2).astype(jnp.float32)
    o_ref[...] = jnp.sin(angles) * (1.0 - cos_mask) + jnp.cos(angles) * cos_mask


def _emb_sc(x, pe, table, batch):
    spw = batch // NW
    nround = spw // NBUF
    mesh = plsc.VectorSubcoreMesh(core_axis_name="c", subcore_axis_name="s")

    @functools.partial(
        pl.kernel,
        out_type=jax.ShapeDtypeStruct((batch, SEQ, DP), jnp.float32),
        mesh=mesh,
        compiler_params=pltpu.CompilerParams(use_tc_tiling_on_sc=False),
        scratch_types=[
            pltpu.VMEM((spw, 2, HALF), jnp.int32),
            pltpu.VMEM((NBUF, SEQ, D), jnp.float32),
            pltpu.VMEM((SEQ, D), jnp.float32),
        ]
        + [pltpu.SemaphoreType.DMA] * (2 * NBUF),
    )
    def k(x_hbm, pe_hbm, table_hbm, out_hbm, idx_v, rows_v, pe_v, *sems):
        gsem = sems[:NBUF]
        osem = sems[NBUF:]
        wid = lax.axis_index("s") * NC + lax.axis_index("c")
        base = wid * spw
        pltpu.sync_copy(pe_hbm, pe_v)
        pltpu.sync_copy(x_hbm.at[pl.ds(base, spw)], idx_v)

        def issue_gather(i, b):
            pltpu.async_copy(
                table_hbm.at[idx_v.at[i, 0]],
                rows_v.at[b, pl.ds(0, HALF)],
                gsem[b],
            )
            pltpu.async_copy(
                table_hbm.at[idx_v.at[i, 1]],
                rows_v.at[b, pl.ds(HALF, HALF)],
                gsem[b],
            )

        def wait_gather(i, b):
            pltpu.make_async_copy(
                table_hbm.at[idx_v.at[i, 0]],
                rows_v.at[b, pl.ds(0, HALF)],
                gsem[b],
            ).wait()
            pltpu.make_async_copy(
                table_hbm.at[idx_v.at[i, 1]],
                rows_v.at[b, pl.ds(HALF, HALF)],
                gsem[b],
            ).wait()

        def issue_out(i, b):
            pltpu.async_copy(
                rows_v.at[b], out_hbm.at[base + i, slice(None), pl.ds(0, D)], osem[b]
            )

        def wait_out(i, b):
            pltpu.make_async_copy(
                rows_v.at[b], out_hbm.at[base + i, slice(None), pl.ds(0, D)], osem[b]
            ).wait()

        issue_gather(0, 0)

        def round_body(r, carry):
            for b in range(NBUF):
                i = r * NBUF + b
                nxt = i + 1
                bn = (b + 1) % NBUF

                @pl.when(jnp.logical_and(nxt >= NBUF, nxt < spw))
                def _():
                    wait_out(nxt - NBUF, bn)

                @pl.when(nxt < spw)
                def _():
                    issue_gather(nxt, bn)

                wait_gather(i, b)

                def jbody(j0, acc):
                    for ju in range(JU):
                        j = j0 * JU + ju
                        for dd in range(D // L):
                            sl = pl.ds(dd * L, L)
                            rows_v[b, j, sl] = rows_v[b, j, sl] + pe_v[j, sl]
                    return acc

                lax.fori_loop(0, SEQ // JU, jbody, 0)

                issue_out(i, b)
            return carry

        lax.fori_loop(0, nround, round_body, 0)
        for b in range(NBUF):
            wait_out(spw - NBUF + b, b)

    return k(x, pe, table)[:, :, :D]


def kernel(x, table):
    batch, seq = x.shape
    assert seq == SEQ and batch % NW == 0
    xi = x.astype(jnp.int32).reshape(batch, 2, HALF)
    pe = pl.pallas_call(
        _pe_tc_kernel,
        out_shape=jax.ShapeDtypeStruct((SEQ, D), jnp.float32),
    )()
    return _emb_sc(xi, pe, table, batch)
